# baseline (speedup 1.0000x reference)
import functools
import numpy as np
import jax
import jax.numpy as jnp
from jax.experimental import pallas as pl

HID = 1024
NH = 16
NKV = 4
HD = 64
E = 8
TOPK = 2
FF = 512
EPS = 1e-06
THETA = 1000000.0

_LOG_THETA = float(np.log(THETA))


def _dot(a, b):
    return jax.lax.dot_general(a.astype(jnp.bfloat16), b.astype(jnp.bfloat16),
                               (((1,), (0,)), ((), ())),
                               preferred_element_type=jnp.float32)


def _dot_t(a, b):
    return jax.lax.dot_general(a.astype(jnp.bfloat16), b.astype(jnp.bfloat16),
                               (((1,), (1,)), ((), ())),
                               preferred_element_type=jnp.float32)


def _rms(x, w, eps=EPS):
    return x * jax.lax.rsqrt(jnp.mean(x * x, axis=-1, keepdims=True) + eps) * w


def _pre_attn_kernel(x_ref, ln_ref, w_ref, qn_ref, kn_ref, q_ref, k_ref, v_ref, *, bt):
    t = pl.program_id(0)
    x = x_ref[...]
    xn = _rms(x, ln_ref[...])
    qkv = _dot(xn, w_ref[...])

    pos = (jax.lax.broadcasted_iota(jnp.int32, (bt, 1), 0) + t * bt).astype(jnp.float32)
    j = jax.lax.broadcasted_iota(jnp.int32, (1, HD // 2), 1).astype(jnp.float32)
    inv = jnp.exp(j * (-2.0 / HD * _LOG_THETA))
    freqs = pos * inv
    cos = jnp.cos(freqs)
    sin = jnp.sin(freqs)

    def rope(xh):
        x1 = xh[:, : HD // 2]
        x2 = xh[:, HD // 2:]
        return jnp.concatenate([x1 * cos - x2 * sin, x2 * cos + x1 * sin], axis=-1)

    qnw = qn_ref[...]
    knw = kn_ref[...]
    for h in range(NH):
        qh = qkv[:, h * HD:(h + 1) * HD]
        q_ref[h] = rope(_rms(qh, qnw))
    for g in range(NKV):
        kh = qkv[:, NH * HD + g * HD: NH * HD + (g + 1) * HD]
        k_ref[g] = rope(_rms(kh, knw))
        v_ref[g] = qkv[:, NH * HD + NKV * HD + g * HD: NH * HD + NKV * HD + (g + 1) * HD]


def _attn_kernel(q_ref, k_ref, v_ref, o_ref, *, bq, T, rep):
    i = pl.program_id(1)
    k = k_ref[0]
    v = v_ref[0]
    row = jax.lax.broadcasted_iota(jnp.int32, (bq, T), 0) + i * bq
    col = jax.lax.broadcasted_iota(jnp.int32, (bq, T), 1)
    causal = col <= row
    outs = []
    for hh in range(rep):
        q = q_ref[hh]
        s = _dot_t(q, k) * (HD ** -0.5)
        s = jnp.where(causal, s, -1e30)
        m = jnp.max(s, axis=-1, keepdims=True)
        p = jnp.exp(s - m)
        p = p / jnp.sum(p, axis=-1, keepdims=True)
        outs.append(_dot(p, v))
    o_ref[...] = jnp.concatenate(outs, axis=-1)


def _post_attn_kernel(o_ref, x_ref, ow_ref, pln_ref, gw_ref,
                      h1_ref, h2_ref, cw_ref):
    h1 = x_ref[...] + _dot(o_ref[...], ow_ref[...])
    h1_ref[...] = h1
    h2 = _rms(h1, pln_ref[...])
    h2_ref[...] = h2
    logits = _dot(h2, gw_ref[...])
    lm = jnp.max(logits, axis=-1, keepdims=True)
    ex = jnp.exp(logits - lm)
    probs = ex / jnp.sum(ex, axis=-1, keepdims=True)
    bt = probs.shape[0]
    lane = jax.lax.broadcasted_iota(jnp.int32, (bt, E), 1)
    m1 = jnp.max(probs, axis=-1, keepdims=True)
    i1 = jnp.min(jnp.where(probs == m1, lane, E), axis=-1, keepdims=True)
    oh1 = lane == i1
    p2 = jnp.where(oh1, -1.0, probs)
    m2 = jnp.max(p2, axis=-1, keepdims=True)
    i2 = jnp.min(jnp.where(p2 == m2, lane, E), axis=-1, keepdims=True)
    oh2 = lane == i2
    denom = m1 + m2
    denom = jnp.where(denom == 0, 1.0, denom)
    cw_ref[...] = (jnp.where(oh1, m1, 0.0) + jnp.where(oh2, m2, 0.0)) / denom


def _moe_kernel(h1_ref, h2_ref, cw_ref, gup_ref, dw_ref, out_ref):
    e = pl.program_id(1)

    @pl.when(e == 0)
    def _():
        out_ref[...] = h1_ref[...]

    h2 = h2_ref[...]
    gu = _dot(h2, gup_ref[0])
    g = gu[:, :FF]
    u = gu[:, FF:]
    act = g * jax.lax.logistic(g) * u
    d = _dot(act, dw_ref[0])
    cw = cw_ref[...]
    lane = jax.lax.broadcasted_iota(jnp.int32, cw.shape, 1)
    w = jnp.sum(jnp.where(lane == e, cw, 0.0), axis=-1, keepdims=True)
    out_ref[...] += d * w


def kernel(hidden_states, positions, input_ln_w, qkv_w, q_norm_w, k_norm_w,
           o_proj_w, post_ln_w, gate_w, gate_up_w, down_w):
    T = hidden_states.shape[0]
    qkv_dim = NH * HD + 2 * NKV * HD

    qkv_wT = qkv_w.T
    o_wT = o_proj_w.T
    gate_wT = gate_w.T
    ln2 = input_ln_w.reshape(1, HID)
    qn2 = q_norm_w.reshape(1, HD)
    kn2 = k_norm_w.reshape(1, HD)
    pln2 = post_ln_w.reshape(1, HID)

    bt = 256
    q, k, v = pl.pallas_call(
        functools.partial(_pre_attn_kernel, bt=bt),
        grid=(T // bt,),
        in_specs=[
            pl.BlockSpec((bt, HID), lambda t: (t, 0)),
            pl.BlockSpec((1, HID), lambda t: (0, 0)),
            pl.BlockSpec((HID, qkv_dim), lambda t: (0, 0)),
            pl.BlockSpec((1, HD), lambda t: (0, 0)),
            pl.BlockSpec((1, HD), lambda t: (0, 0)),
        ],
        out_specs=[
            pl.BlockSpec((NH, bt, HD), lambda t: (0, t, 0)),
            pl.BlockSpec((NKV, bt, HD), lambda t: (0, t, 0)),
            pl.BlockSpec((NKV, bt, HD), lambda t: (0, t, 0)),
        ],
        out_shape=[
            jax.ShapeDtypeStruct((NH, T, HD), jnp.float32),
            jax.ShapeDtypeStruct((NKV, T, HD), jnp.float32),
            jax.ShapeDtypeStruct((NKV, T, HD), jnp.float32),
        ],
    )(hidden_states, ln2, qkv_wT, qn2, kn2)

    bq = 256
    rep = NH // NKV
    o = pl.pallas_call(
        functools.partial(_attn_kernel, bq=bq, T=T, rep=rep),
        grid=(NKV, T // bq),
        in_specs=[
            pl.BlockSpec((rep, bq, HD), lambda g, i: (g, i, 0)),
            pl.BlockSpec((1, T, HD), lambda g, i: (g, 0, 0)),
            pl.BlockSpec((1, T, HD), lambda g, i: (g, 0, 0)),
        ],
        out_specs=pl.BlockSpec((bq, rep * HD), lambda g, i: (i, g)),
        out_shape=jax.ShapeDtypeStruct((T, NH * HD), jnp.float32),
    )(q, k, v)

    h1, h2, cw = pl.pallas_call(
        _post_attn_kernel,
        grid=(T // bt,),
        in_specs=[
            pl.BlockSpec((bt, NH * HD), lambda t: (t, 0)),
            pl.BlockSpec((bt, HID), lambda t: (t, 0)),
            pl.BlockSpec((NH * HD, HID), lambda t: (0, 0)),
            pl.BlockSpec((1, HID), lambda t: (0, 0)),
            pl.BlockSpec((HID, E), lambda t: (0, 0)),
        ],
        out_specs=[
            pl.BlockSpec((bt, HID), lambda t: (t, 0)),
            pl.BlockSpec((bt, HID), lambda t: (t, 0)),
            pl.BlockSpec((bt, E), lambda t: (t, 0)),
        ],
        out_shape=[
            jax.ShapeDtypeStruct((T, HID), jnp.float32),
            jax.ShapeDtypeStruct((T, HID), jnp.float32),
            jax.ShapeDtypeStruct((T, E), jnp.float32),
        ],
    )(o, hidden_states, o_wT, pln2, gate_wT)

    bm = min(1024, T)
    out = pl.pallas_call(
        _moe_kernel,
        grid=(T // bm, E),
        in_specs=[
            pl.BlockSpec((bm, HID), lambda t, e: (t, 0)),
            pl.BlockSpec((bm, HID), lambda t, e: (t, 0)),
            pl.BlockSpec((bm, E), lambda t, e: (t, 0)),
            pl.BlockSpec((1, HID, 2 * FF), lambda t, e: (e, 0, 0)),
            pl.BlockSpec((1, FF, HID), lambda t, e: (e, 0, 0)),
        ],
        out_specs=pl.BlockSpec((bm, HID), lambda t, e: (t, 0)),
        out_shape=jax.ShapeDtypeStruct((T, HID), jnp.float32),
    )(h1, h2, cw, gate_up_w, down_w)

    return out


# speedup vs baseline: 1.1529x; 1.1529x over previous
import functools
import numpy as np
import jax
import jax.numpy as jnp
from jax.experimental import pallas as pl

HID = 1024
NH = 16
NKV = 4
HD = 64
E = 8
TOPK = 2
FF = 512
EPS = 1e-06
THETA = 1000000.0

_LOG_THETA = float(np.log(THETA))


def _dot(a, b):
    return jax.lax.dot_general(a, b, (((1,), (0,)), ((), ())),
                               preferred_element_type=jnp.float32)


def _dot_t(a, b):
    return jax.lax.dot_general(a, b, (((1,), (1,)), ((), ())),
                               preferred_element_type=jnp.float32)


def _rms(x, w, eps=EPS):
    return x * jax.lax.rsqrt(jnp.mean(x * x, axis=-1, keepdims=True) + eps) * w


def _pre_attn_kernel(x_ref, ln_ref, w_ref, qn_ref, kn_ref, q_ref, k_ref, v_ref, *, bt):
    t = pl.program_id(0)
    x = x_ref[...]
    xn = _rms(x, ln_ref[...])
    qkv = _dot(xn, w_ref[...])

    pos = (jax.lax.broadcasted_iota(jnp.int32, (bt, 1), 0) + t * bt).astype(jnp.float32)
    j = jax.lax.broadcasted_iota(jnp.int32, (1, HD // 2), 1).astype(jnp.float32)
    inv = jnp.exp(j * (-2.0 / HD * _LOG_THETA))
    freqs = pos * inv
    cos = jnp.cos(freqs)
    sin = jnp.sin(freqs)

    def rope(xh):
        x1 = xh[:, : HD // 2]
        x2 = xh[:, HD // 2:]
        return jnp.concatenate([x1 * cos - x2 * sin, x2 * cos + x1 * sin], axis=-1)

    qnw = qn_ref[...]
    knw = kn_ref[...]
    for h in range(NH):
        qh = qkv[:, h * HD:(h + 1) * HD]
        q_ref[h] = rope(_rms(qh, qnw))
    for g in range(NKV):
        kh = qkv[:, NH * HD + g * HD: NH * HD + (g + 1) * HD]
        k_ref[g] = rope(_rms(kh, knw))
        v_ref[g] = qkv[:, NH * HD + NKV * HD + g * HD: NH * HD + NKV * HD + (g + 1) * HD]


def _attn_kernel(q_ref, k_ref, v_ref, o_ref, *, bq, T, rep):
    i = pl.program_id(1)
    scale = HD ** -0.5
    qs = [q_ref[hh] * scale for hh in range(rep)]

    def body(j, carry):
        accs, ls = carry
        kj = k_ref[0, pl.ds(j * bq, bq), :]
        vj = v_ref[0, pl.ds(j * bq, bq), :]
        new_accs = []
        new_ls = []
        for hh in range(rep):
            p = jnp.exp(_dot_t(qs[hh], kj))
            new_ls.append(ls[hh] + jnp.sum(p, axis=-1, keepdims=True))
            new_accs.append(accs[hh] + _dot(p, vj))
        return new_accs, new_ls

    init = ([jnp.zeros((bq, HD), jnp.float32) for _ in range(rep)],
            [jnp.zeros((bq, 1), jnp.float32) for _ in range(rep)])
    accs, ls = jax.lax.fori_loop(0, i, body, init)

    kd = k_ref[0, pl.ds(i * bq, bq), :]
    vd = v_ref[0, pl.ds(i * bq, bq), :]
    rowl = jax.lax.broadcasted_iota(jnp.int32, (bq, bq), 0)
    coll = jax.lax.broadcasted_iota(jnp.int32, (bq, bq), 1)
    causal = coll <= rowl
    outs = []
    for hh in range(rep):
        p = jnp.where(causal, jnp.exp(_dot_t(qs[hh], kd)), 0.0)
        l = ls[hh] + jnp.sum(p, axis=-1, keepdims=True)
        acc = accs[hh] + _dot(p, vd)
        outs.append(acc / l)
    o_ref[...] = jnp.concatenate(outs, axis=-1)


def _post_attn_kernel(o_ref, x_ref, ow_ref, pln_ref, gw_ref,
                      h1_ref, h2_ref, cw_ref):
    h1 = x_ref[...] + _dot(o_ref[...], ow_ref[...])
    h1_ref[...] = h1
    h2 = _rms(h1, pln_ref[...])
    h2_ref[...] = h2
    logits = _dot(h2, gw_ref[...])
    lm = jnp.max(logits, axis=-1, keepdims=True)
    ex = jnp.exp(logits - lm)
    probs = ex / jnp.sum(ex, axis=-1, keepdims=True)
    bt = probs.shape[0]
    lane = jax.lax.broadcasted_iota(jnp.int32, (bt, E), 1)
    m1 = jnp.max(probs, axis=-1, keepdims=True)
    i1 = jnp.min(jnp.where(probs == m1, lane, E), axis=-1, keepdims=True)
    oh1 = lane == i1
    p2 = jnp.where(oh1, -1.0, probs)
    m2 = jnp.max(p2, axis=-1, keepdims=True)
    i2 = jnp.min(jnp.where(p2 == m2, lane, E), axis=-1, keepdims=True)
    oh2 = lane == i2
    denom = m1 + m2
    denom = jnp.where(denom == 0, 1.0, denom)
    cw_ref[...] = (jnp.where(oh1, m1, 0.0) + jnp.where(oh2, m2, 0.0)) / denom


def _moe_kernel(h1_ref, h2_ref, cw_ref, gup_ref, dw_ref, out_ref):
    e = pl.program_id(1)

    @pl.when(e == 0)
    def _():
        out_ref[...] = h1_ref[...]

    h2 = h2_ref[...]
    gu = _dot(h2, gup_ref[0])
    g = gu[:, :FF]
    u = gu[:, FF:]
    act = g * jax.lax.logistic(g) * u
    d = _dot(act, dw_ref[0])
    cw = cw_ref[...]
    lane = jax.lax.broadcasted_iota(jnp.int32, cw.shape, 1)
    w = jnp.sum(jnp.where(lane == e, cw, 0.0), axis=-1, keepdims=True)
    out_ref[...] += d * w


def kernel(hidden_states, positions, input_ln_w, qkv_w, q_norm_w, k_norm_w,
           o_proj_w, post_ln_w, gate_w, gate_up_w, down_w):
    T = hidden_states.shape[0]
    qkv_dim = NH * HD + 2 * NKV * HD

    qkv_wT = qkv_w.T
    o_wT = o_proj_w.T
    gate_wT = gate_w.T
    ln2 = input_ln_w.reshape(1, HID)
    qn2 = q_norm_w.reshape(1, HD)
    kn2 = k_norm_w.reshape(1, HD)
    pln2 = post_ln_w.reshape(1, HID)

    bt = 256
    q, k, v = pl.pallas_call(
        functools.partial(_pre_attn_kernel, bt=bt),
        grid=(T // bt,),
        in_specs=[
            pl.BlockSpec((bt, HID), lambda t: (t, 0)),
            pl.BlockSpec((1, HID), lambda t: (0, 0)),
            pl.BlockSpec((HID, qkv_dim), lambda t: (0, 0)),
            pl.BlockSpec((1, HD), lambda t: (0, 0)),
            pl.BlockSpec((1, HD), lambda t: (0, 0)),
        ],
        out_specs=[
            pl.BlockSpec((NH, bt, HD), lambda t: (0, t, 0)),
            pl.BlockSpec((NKV, bt, HD), lambda t: (0, t, 0)),
            pl.BlockSpec((NKV, bt, HD), lambda t: (0, t, 0)),
        ],
        out_shape=[
            jax.ShapeDtypeStruct((NH, T, HD), jnp.float32),
            jax.ShapeDtypeStruct((NKV, T, HD), jnp.float32),
            jax.ShapeDtypeStruct((NKV, T, HD), jnp.float32),
        ],
    )(hidden_states, ln2, qkv_wT, qn2, kn2)

    bq = 256
    rep = NH // NKV
    o = pl.pallas_call(
        functools.partial(_attn_kernel, bq=bq, T=T, rep=rep),
        grid=(NKV, T // bq),
        in_specs=[
            pl.BlockSpec((rep, bq, HD), lambda g, i: (g, i, 0)),
            pl.BlockSpec((1, T, HD), lambda g, i: (g, 0, 0)),
            pl.BlockSpec((1, T, HD), lambda g, i: (g, 0, 0)),
        ],
        out_specs=pl.BlockSpec((bq, rep * HD), lambda g, i: (i, g)),
        out_shape=jax.ShapeDtypeStruct((T, NH * HD), jnp.float32),
    )(q, k, v)

    h1, h2, cw = pl.pallas_call(
        _post_attn_kernel,
        grid=(T // bt,),
        in_specs=[
            pl.BlockSpec((bt, NH * HD), lambda t: (t, 0)),
            pl.BlockSpec((bt, HID), lambda t: (t, 0)),
            pl.BlockSpec((NH * HD, HID), lambda t: (0, 0)),
            pl.BlockSpec((1, HID), lambda t: (0, 0)),
            pl.BlockSpec((HID, E), lambda t: (0, 0)),
        ],
        out_specs=[
            pl.BlockSpec((bt, HID), lambda t: (t, 0)),
            pl.BlockSpec((bt, HID), lambda t: (t, 0)),
            pl.BlockSpec((bt, E), lambda t: (t, 0)),
        ],
        out_shape=[
            jax.ShapeDtypeStruct((T, HID), jnp.float32),
            jax.ShapeDtypeStruct((T, HID), jnp.float32),
            jax.ShapeDtypeStruct((T, E), jnp.float32),
        ],
    )(o, hidden_states, o_wT, pln2, gate_wT)

    bm = min(1024, T)
    out = pl.pallas_call(
        _moe_kernel,
        grid=(T // bm, E),
        in_specs=[
            pl.BlockSpec((bm, HID), lambda t, e: (t, 0)),
            pl.BlockSpec((bm, HID), lambda t, e: (t, 0)),
            pl.BlockSpec((bm, E), lambda t, e: (t, 0)),
            pl.BlockSpec((1, HID, 2 * FF), lambda t, e: (e, 0, 0)),
            pl.BlockSpec((1, FF, HID), lambda t, e: (e, 0, 0)),
        ],
        out_specs=pl.BlockSpec((bm, HID), lambda t, e: (t, 0)),
        out_shape=jax.ShapeDtypeStruct((T, HID), jnp.float32),
    )(h1, h2, cw, gate_up_w, down_w)

    return out


# speedup vs baseline: 1.2357x; 1.0718x over previous
import functools
import numpy as np
import jax
import jax.numpy as jnp
from jax.experimental import pallas as pl
import jax.experimental.pallas.tpu as pltpu

HID = 1024
NH = 16
NKV = 4
HD = 64
E = 8
TOPK = 2
FF = 512
EPS = 1e-06
THETA = 1000000.0

_LOG_THETA = float(np.log(THETA))
_HALF = HD // 2


def _dot(a, b):
    return jax.lax.dot_general(a, b, (((1,), (0,)), ((), ())),
                               preferred_element_type=jnp.float32)


def _dot_t(a, b):
    return jax.lax.dot_general(a, b, (((1,), (1,)), ((), ())),
                               preferred_element_type=jnp.float32)


def _dot_tl(a, b):
    return jax.lax.dot_general(a, b, (((1,), (1,)), ((), ())),
                               preferred_element_type=jnp.float32)


def _rms(x, w, eps=EPS):
    return x * jax.lax.rsqrt(jnp.mean(x * x, axis=-1, keepdims=True) + eps) * w


def _seg(nheads):
    n = nheads * _HALF
    l = jax.lax.broadcasted_iota(jnp.int32, (n, nheads), 0)
    h = jax.lax.broadcasted_iota(jnp.int32, (n, nheads), 1)
    return (l // _HALF == h).astype(jnp.float32)


def _tile_mat(nheads):
    n = nheads * _HALF
    j = jax.lax.broadcasted_iota(jnp.int32, (_HALF, n), 0)
    l = jax.lax.broadcasted_iota(jnp.int32, (_HALF, n), 1)
    return (l % _HALF == j).astype(jnp.float32)


def _pre_attn_kernel(x_ref, ln_ref, w_ref, qn1_ref, qn2_ref, kn1_ref, kn2_ref,
                     q_ref, k_ref, v_ref, *, bt):
    t = pl.program_id(0)
    x = x_ref[...]
    xn = _rms(x, ln_ref[...])
    qkv = _dot(xn, w_ref[...])

    pos = (jax.lax.broadcasted_iota(jnp.int32, (bt, 1), 0) + t * bt).astype(jnp.float32)
    j = jax.lax.broadcasted_iota(jnp.int32, (1, _HALF), 1).astype(jnp.float32)
    inv = jnp.exp(j * (-2.0 / HD * _LOG_THETA))
    freqs = pos * inv
    cos = jnp.cos(freqs)
    sin = jnp.sin(freqs)

    nq = NH * _HALF
    nk = NKV * _HALF
    q1 = qkv[:, :nq]
    q2 = qkv[:, nq:2 * nq]
    k1 = qkv[:, 2 * nq:2 * nq + nk]
    k2 = qkv[:, 2 * nq + nk:2 * nq + 2 * nk]
    v = qkv[:, 2 * nq + 2 * nk:]

    segq = _seg(NH)
    ssq = _dot(q1 * q1 + q2 * q2, segq)
    rstd = jax.lax.rsqrt(ssq * (1.0 / HD) + EPS)
    rstd_w = _dot_tl(rstd, segq)
    tq = _tile_mat(NH)
    cos_q = _dot(cos, tq)
    sin_q = _dot(sin, tq)
    q1n = q1 * rstd_w * qn1_ref[...]
    q2n = q2 * rstd_w * qn2_ref[...]
    q_ref[...] = jnp.concatenate([q1n * cos_q - q2n * sin_q,
                                  q2n * cos_q + q1n * sin_q], axis=1)

    segk = _seg(NKV)
    ssk = _dot(k1 * k1 + k2 * k2, segk)
    rstdk = jax.lax.rsqrt(ssk * (1.0 / HD) + EPS)
    rstdk_w = _dot_tl(rstdk, segk)
    tk = _tile_mat(NKV)
    cos_k = _dot(cos, tk)
    sin_k = _dot(sin, tk)
    k1n = k1 * rstdk_w * kn1_ref[...]
    k2n = k2 * rstdk_w * kn2_ref[...]
    k1r = k1n * cos_k - k2n * sin_k
    k2r = k2n * cos_k + k1n * sin_k
    for g in range(NKV):
        k_ref[g] = jnp.concatenate(
            [k1r[:, g * _HALF:(g + 1) * _HALF], k2r[:, g * _HALF:(g + 1) * _HALF]], axis=1)
        v_ref[g] = v[:, g * HD:(g + 1) * HD]


def _attn_kernel(qa_ref, qb_ref, k_ref, v_ref, o_ref, *, bq, T, rep):
    i = pl.program_id(1)
    scale = HD ** -0.5
    qs = [jnp.concatenate([qa_ref[:, hh * _HALF:(hh + 1) * _HALF],
                           qb_ref[:, hh * _HALF:(hh + 1) * _HALF]], axis=1) * scale
          for hh in range(rep)]

    def body(j, carry):
        accs, ls = carry
        kj = k_ref[0, pl.ds(j * bq, bq), :]
        vj = v_ref[0, pl.ds(j * bq, bq), :]
        new_accs = []
        new_ls = []
        for hh in range(rep):
            p = jnp.exp(_dot_t(qs[hh], kj))
            new_ls.append(ls[hh] + jnp.sum(p, axis=-1, keepdims=True))
            new_accs.append(accs[hh] + _dot(p, vj))
        return new_accs, new_ls

    init = ([jnp.zeros((bq, HD), jnp.float32) for _ in range(rep)],
            [jnp.zeros((bq, 1), jnp.float32) for _ in range(rep)])
    accs, ls = jax.lax.fori_loop(0, i, body, init)

    kd = k_ref[0, pl.ds(i * bq, bq), :]
    vd = v_ref[0, pl.ds(i * bq, bq), :]
    rowl = jax.lax.broadcasted_iota(jnp.int32, (bq, bq), 0)
    coll = jax.lax.broadcasted_iota(jnp.int32, (bq, bq), 1)
    causal = coll <= rowl
    outs = []
    for hh in range(rep):
        p = jnp.where(causal, jnp.exp(_dot_t(qs[hh], kd)), 0.0)
        l = ls[hh] + jnp.sum(p, axis=-1, keepdims=True)
        acc = accs[hh] + _dot(p, vd)
        outs.append(acc / l)
    o_ref[...] = jnp.concatenate(outs, axis=-1)


def _moe_kernel(o_ref, x_ref, ow_ref, pln_ref, gw_ref, gup_ref, dw_ref,
                out_ref, h2_s, cw_s):
    e = pl.program_id(1)

    @pl.when(e == 0)
    def _():
        h1 = x_ref[...] + _dot(o_ref[...], ow_ref[...])
        out_ref[...] = h1
        h2 = _rms(h1, pln_ref[...])
        h2_s[...] = h2
        logits = _dot(h2, gw_ref[...])
        lm = jnp.max(logits, axis=-1, keepdims=True)
        ex = jnp.exp(logits - lm)
        probs = ex / jnp.sum(ex, axis=-1, keepdims=True)
        bt = probs.shape[0]
        lane = jax.lax.broadcasted_iota(jnp.int32, (bt, E), 1)
        m1 = jnp.max(probs, axis=-1, keepdims=True)
        i1 = jnp.min(jnp.where(probs == m1, lane, E), axis=-1, keepdims=True)
        oh1 = lane == i1
        p2 = jnp.where(oh1, -1.0, probs)
        m2 = jnp.max(p2, axis=-1, keepdims=True)
        i2 = jnp.min(jnp.where(p2 == m2, lane, E), axis=-1, keepdims=True)
        oh2 = lane == i2
        denom = m1 + m2
        denom = jnp.where(denom == 0, 1.0, denom)
        cw_s[...] = (jnp.where(oh1, m1, 0.0) + jnp.where(oh2, m2, 0.0)) / denom

    h2 = h2_s[...]
    gu = _dot(h2, gup_ref[0])
    g = gu[:, :FF]
    u = gu[:, FF:]
    act = g * jax.lax.logistic(g) * u
    d = _dot(act, dw_ref[0])
    cw = cw_s[...]
    lane = jax.lax.broadcasted_iota(jnp.int32, cw.shape, 1)
    w = jnp.sum(jnp.where(lane == e, cw, 0.0), axis=-1, keepdims=True)
    out_ref[...] += d * w


def kernel(hidden_states, positions, input_ln_w, qkv_w, q_norm_w, k_norm_w,
           o_proj_w, post_ln_w, gate_w, gate_up_w, down_w):
    T = hidden_states.shape[0]
    qkv_dim = NH * HD + 2 * NKV * HD

    qperm = np.concatenate([
        np.concatenate([np.arange(h * HD, h * HD + _HALF) for h in range(NH)]),
        np.concatenate([np.arange(h * HD + _HALF, (h + 1) * HD) for h in range(NH)]),
        np.concatenate([np.arange(NH * HD + g * HD, NH * HD + g * HD + _HALF) for g in range(NKV)]),
        np.concatenate([np.arange(NH * HD + g * HD + _HALF, NH * HD + (g + 1) * HD) for g in range(NKV)]),
        np.arange(NH * HD + NKV * HD, qkv_dim),
    ])
    qkv_wT = qkv_w.T[:, qperm]
    o_wT = o_proj_w.T
    gate_wT = gate_w.T
    ln2 = input_ln_w.reshape(1, HID)
    qn1 = jnp.tile(q_norm_w[:_HALF], NH).reshape(1, NH * _HALF)
    qn2 = jnp.tile(q_norm_w[_HALF:], NH).reshape(1, NH * _HALF)
    kn1 = jnp.tile(k_norm_w[:_HALF], NKV).reshape(1, NKV * _HALF)
    kn2 = jnp.tile(k_norm_w[_HALF:], NKV).reshape(1, NKV * _HALF)
    pln2 = post_ln_w.reshape(1, HID)

    bt = 256
    q, k, v = pl.pallas_call(
        functools.partial(_pre_attn_kernel, bt=bt),
        grid=(T // bt,),
        in_specs=[
            pl.BlockSpec((bt, HID), lambda t: (t, 0)),
            pl.BlockSpec((1, HID), lambda t: (0, 0)),
            pl.BlockSpec((HID, qkv_dim), lambda t: (0, 0)),
            pl.BlockSpec((1, NH * _HALF), lambda t: (0, 0)),
            pl.BlockSpec((1, NH * _HALF), lambda t: (0, 0)),
            pl.BlockSpec((1, NKV * _HALF), lambda t: (0, 0)),
            pl.BlockSpec((1, NKV * _HALF), lambda t: (0, 0)),
        ],
        out_specs=[
            pl.BlockSpec((bt, NH * HD), lambda t: (t, 0)),
            pl.BlockSpec((NKV, bt, HD), lambda t: (0, t, 0)),
            pl.BlockSpec((NKV, bt, HD), lambda t: (0, t, 0)),
        ],
        out_shape=[
            jax.ShapeDtypeStruct((T, NH * HD), jnp.float32),
            jax.ShapeDtypeStruct((NKV, T, HD), jnp.float32),
            jax.ShapeDtypeStruct((NKV, T, HD), jnp.float32),
        ],
    )(hidden_states, ln2, qkv_wT, qn1, qn2, kn1, kn2)

    bq = 256
    rep = NH // NKV
    o = pl.pallas_call(
        functools.partial(_attn_kernel, bq=bq, T=T, rep=rep),
        grid=(NKV, T // bq),
        in_specs=[
            pl.BlockSpec((bq, rep * _HALF), lambda g, i: (i, g)),
            pl.BlockSpec((bq, rep * _HALF), lambda g, i: (i, NKV + g)),
            pl.BlockSpec((1, T, HD), lambda g, i: (g, 0, 0)),
            pl.BlockSpec((1, T, HD), lambda g, i: (g, 0, 0)),
        ],
        out_specs=pl.BlockSpec((bq, rep * HD), lambda g, i: (i, g)),
        out_shape=jax.ShapeDtypeStruct((T, NH * HD), jnp.float32),
    )(q, q, k, v)

    bm = min(1024, T)
    out = pl.pallas_call(
        _moe_kernel,
        grid=(T // bm, E),
        in_specs=[
            pl.BlockSpec((bm, NH * HD), lambda t, e: (t, 0)),
            pl.BlockSpec((bm, HID), lambda t, e: (t, 0)),
            pl.BlockSpec((NH * HD, HID), lambda t, e: (0, 0)),
            pl.BlockSpec((1, HID), lambda t, e: (0, 0)),
            pl.BlockSpec((HID, E), lambda t, e: (0, 0)),
            pl.BlockSpec((1, HID, 2 * FF), lambda t, e: (e, 0, 0)),
            pl.BlockSpec((1, FF, HID), lambda t, e: (e, 0, 0)),
        ],
        out_specs=pl.BlockSpec((bm, HID), lambda t, e: (t, 0)),
        out_shape=jax.ShapeDtypeStruct((T, HID), jnp.float32),
        scratch_shapes=[
            pltpu.VMEM((bm, HID), jnp.float32),
            pltpu.VMEM((bm, E), jnp.float32),
        ],
    )(o, hidden_states, o_wT, pln2, gate_wT, gate_up_w, down_w)

    return out
